# Initial kernel scaffold; baseline (speedup 1.0000x reference)
import jax
import jax.numpy as jnp
from jax.experimental import pallas as pl


def kernel(x, edge_index, W0, a_src0, a_dst0, b0, W1, a_src1, a_dst1, b1, W2, a_src2, a_dst2, b2, W_out, b_out):
    raise NotImplementedError("write your pallas kernel here")


# speedup vs baseline: 16.9550x; 16.9550x over previous
import functools

import jax
import jax.numpy as jnp
from jax import lax
from jax.experimental import pallas as pl
from jax.experimental.pallas import tpu as pltpu
from jax.experimental.pallas import tpu_sc as plsc

N = 10000
NP = 10240
E = 320000
EP = 327680
DH = 256
DHALF = 128
NCLS = 40

NC = 2
NS = 16
NW = NC * NS

ER = EP // 128


BN = 1024


def _elu(x):
    return jnp.where(x > 0, x, jnp.exp(x) - 1.0)


def _tc_layer0_body(x_ref, w_ref, a2_ref, hs_ref, asad_ref):
    h = jnp.dot(x_ref[...], w_ref[...], preferred_element_type=jnp.float32)
    hs_ref[0] = h[:, :DHALF]
    hs_ref[1] = h[:, DHALF:]
    asad_ref[...] = lax.dot_general(a2_ref[...], h, (((1,), (1,)), ((), ())))


def _tc_layer_body(p_ref, b_ref, w_ref, a2_ref, hs_ref, asad_ref):
    xin = _elu(jnp.concatenate([p_ref[0], p_ref[1]], axis=-1) + b_ref[...])
    h = jnp.dot(xin, w_ref[...], preferred_element_type=jnp.float32)
    hs_ref[0] = h[:, :DHALF]
    hs_ref[1] = h[:, DHALF:]
    asad_ref[...] = lax.dot_general(a2_ref[...], h, (((1,), (1,)), ((), ())))


_TC_OUT_SHAPE = [
    jax.ShapeDtypeStruct((2, NP, DHALF), jnp.float32),
    jax.ShapeDtypeStruct((2, NP), jnp.float32),
]
_TC_OUT_SPECS = [
    pl.BlockSpec((2, BN, DHALF), lambda i: (0, i, 0)),
    pl.BlockSpec((2, BN), lambda i: (0, i)),
]


def _tc_layer0(xp, W, a2):
    return pl.pallas_call(
        _tc_layer0_body,
        grid=(NP // BN,),
        in_specs=[
            pl.BlockSpec((BN, 128), lambda i: (i, 0)),
            pl.BlockSpec((128, DH), lambda i: (0, 0)),
            pl.BlockSpec((2, DH), lambda i: (0, 0)),
        ],
        out_specs=_TC_OUT_SPECS,
        out_shape=_TC_OUT_SHAPE,
    )(xp, W, a2)


def _tc_layer(prev_split, b_prev, W, a2):
    return pl.pallas_call(
        _tc_layer_body,
        grid=(NP // BN,),
        in_specs=[
            pl.BlockSpec((2, BN, DHALF), lambda i: (0, i, 0)),
            pl.BlockSpec((1, DH), lambda i: (0, 0)),
            pl.BlockSpec((DH, DH), lambda i: (0, 0)),
            pl.BlockSpec((2, DH), lambda i: (0, 0)),
        ],
        out_specs=_TC_OUT_SPECS,
        out_shape=_TC_OUT_SHAPE,
    )(prev_split, b_prev, W, a2)


def _tc_final_body(a0_ref, a1_ref, a2_ref, b3_ref, wo_ref, bo_ref, out_ref):
    parts = []
    for l, ar in enumerate((a0_ref, a1_ref, a2_ref)):
        xl = jnp.concatenate([ar[0], ar[1]], axis=-1) + b3_ref[pl.ds(l, 1), :]
        parts.append(_elu(xl))
    cat = jnp.concatenate(parts, axis=-1)
    out_ref[...] = (
        jnp.dot(cat, wo_ref[...], preferred_element_type=jnp.float32) + bo_ref[...]
    )


def _tc_final(agg0, agg1, agg2, b3, wo_pad, bo_pad):
    return pl.pallas_call(
        _tc_final_body,
        grid=(NP // BN,),
        in_specs=[
            pl.BlockSpec((2, BN, DHALF), lambda i: (0, i, 0)),
            pl.BlockSpec((2, BN, DHALF), lambda i: (0, i, 0)),
            pl.BlockSpec((2, BN, DHALF), lambda i: (0, i, 0)),
            pl.BlockSpec((3, DH), lambda i: (0, 0)),
            pl.BlockSpec((3 * DH, 128), lambda i: (0, 0)),
            pl.BlockSpec((1, 128), lambda i: (0, 0)),
        ],
        out_specs=pl.BlockSpec((BN, 128), lambda i: (i, 0)),
        out_shape=jax.ShapeDtypeStruct((NP, 128), jnp.float32),
    )(agg0, agg1, agg2, b3, wo_pad, bo_pad)


_MESH = plsc.VectorSubcoreMesh(core_axis_name="c", subcore_axis_name="s")

EW_A = EP // NW
NSUB = 8
NCHA = EW_A // (NSUB * 128)
NTS = NP // NS


def _sc_edge_softmax(src2d, dst2d, as_arr, ad_arr):
    @functools.partial(
        pl.kernel,
        out_type=[
            jax.ShapeDtypeStruct((ER, 128), jnp.float32),
            jax.ShapeDtypeStruct((NC * NP,), jnp.float32),
        ],
        mesh=_MESH,
        scratch_types=[
            pltpu.VMEM((NSUB, 128), jnp.int32),
            pltpu.VMEM((NSUB, 128), jnp.int32),
            pltpu.VMEM((NSUB, 128), jnp.float32),
            pltpu.VMEM((NSUB, 128), jnp.float32),
            pltpu.VMEM((NSUB, 128), jnp.float32),
            pltpu.VMEM((NTS,), jnp.float32),
            pltpu.VMEM_SHARED((NP,), jnp.float32),
        ],
    )
    def k(src_hbm, dst_hbm, as_hbm, ad_hbm, exw_hbm, den_hbm,
          src_v, dst_v, as_v, ad_v, ex_v, stage_v, den_sh):
        c = lax.axis_index("c")
        s = lax.axis_index("s")
        wid = c * NS + s

        @pl.loop(0, NTS // 16)
        def _zero(i):
            stage_v[pl.ds(i * 16, 16)] = jnp.zeros((16,), jnp.float32)

        pltpu.sync_copy(stage_v, den_sh.at[pl.ds(s * NTS, NTS)])
        plsc.subcore_barrier()

        rbase = wid * (EW_A // 128)

        @pl.loop(0, NCHA)
        def _chunk(g):
            roff = rbase + g * NSUB
            pltpu.sync_copy(src_hbm.at[pl.ds(roff, NSUB)], src_v)
            pltpu.sync_copy(dst_hbm.at[pl.ds(roff, NSUB)], dst_v)
            for j in range(NSUB):
                pltpu.sync_copy(as_hbm.at[src_v.at[j]], as_v.at[j])
                pltpu.sync_copy(ad_hbm.at[dst_v.at[j]], ad_v.at[j])

            @pl.loop(0, NSUB)
            def _vec(j):
                for kk in range(8):
                    sl = pl.ds(kk * 16, 16)
                    e = as_v[j, sl] + ad_v[j, sl]
                    e = jnp.where(e >= 0.0, e, e * jnp.float32(0.2))
                    ex_v[j, sl] = jnp.exp(e)

            pltpu.sync_copy(ex_v, exw_hbm.at[pl.ds(roff, NSUB)])
            for j in range(NSUB):
                pltpu.sync_copy(ex_v.at[j], den_sh.at[dst_v.at[j]], add=True)

        plsc.subcore_barrier()
        pltpu.sync_copy(den_sh.at[pl.ds(s * NTS, NTS)], stage_v)
        pltpu.sync_copy(stage_v, den_hbm.at[pl.ds(c * NP + s * NTS, NTS)])

    return k(src2d, dst2d, as_arr, ad_arr)


def _sc_edge_weights(dst2d, exw2d, den2):
    @functools.partial(
        pl.kernel,
        out_type=jax.ShapeDtypeStruct((ER, 128), jnp.float32),
        mesh=_MESH,
        scratch_types=[
            pltpu.VMEM((NSUB, 128), jnp.int32),
            pltpu.VMEM((NSUB, 128), jnp.int32),
            pltpu.VMEM((NSUB, 128), jnp.float32),
            pltpu.VMEM((NSUB, 128), jnp.float32),
            pltpu.VMEM((NSUB, 128), jnp.float32),
        ],
    )
    def k(dst_hbm, exw_hbm, den_hbm, w_hbm, dst_v, dstp_v, ex_v, d0_v, d1_v):
        c = lax.axis_index("c")
        s = lax.axis_index("s")
        wid = c * NS + s
        rbase = wid * (EW_A // 128)

        @pl.loop(0, NCHA)
        def _chunk(g):
            roff = rbase + g * NSUB
            pltpu.sync_copy(dst_hbm.at[pl.ds(roff, NSUB)], dst_v)
            pltpu.sync_copy(exw_hbm.at[pl.ds(roff, NSUB)], ex_v)

            @pl.loop(0, NSUB)
            def _shift(j):
                for kk in range(8):
                    sl = pl.ds(kk * 16, 16)
                    dstp_v[j, sl] = dst_v[j, sl] + NP

            for j in range(NSUB):
                pltpu.sync_copy(den_hbm.at[dst_v.at[j]], d0_v.at[j])
                pltpu.sync_copy(den_hbm.at[dstp_v.at[j]], d1_v.at[j])

            @pl.loop(0, NSUB)
            def _vec(j):
                for kk in range(8):
                    sl = pl.ds(kk * 16, 16)
                    den = d0_v[j, sl] + d1_v[j, sl] + jnp.float32(1e-16)
                    ex_v[j, sl] = ex_v[j, sl] / den

            pltpu.sync_copy(ex_v, w_hbm.at[pl.ds(roff, NSUB)])

    return k(dst2d, exw2d, den2)


EW_B = EP // NS
GB = EW_B // 128


def _sc_aggregate(src2d, dst2d, w2d, h_split):
    @functools.partial(
        pl.kernel,
        out_type=jax.ShapeDtypeStruct((2, NP, DHALF), jnp.float32),
        mesh=_MESH,
        scratch_types=[
            pltpu.VMEM((8, 128), jnp.int32),
            pltpu.VMEM((8, 128), jnp.int32),
            pltpu.VMEM((8, 128), jnp.float32),
            pltpu.VMEM((128, DHALF), jnp.float32),
            pltpu.VMEM((128, DHALF), jnp.float32),
            pltpu.VMEM_SHARED((NP, DHALF), jnp.float32),
            pltpu.SemaphoreType.DMA,
            pltpu.SemaphoreType.DMA,
        ],
    )
    def k(src_hbm, dst_hbm, w_hbm, hs_hbm, out_hbm,
          sc_v, dd_v, wb_v, rows0, rows1, out_sh, sem0, sem1):
        c = lax.axis_index("c")
        s = lax.axis_index("s")

        @pl.loop(0, 128)
        def _zr(r):
            for kk in range(DHALF // 16):
                rows0[r, pl.ds(kk * 16, 16)] = jnp.zeros((16,), jnp.float32)

        for t in range(NTS // 128):
            pltpu.sync_copy(rows0, out_sh.at[pl.ds(s * NTS + t * 128, 128)])
        plsc.subcore_barrier()

        h_half = hs_hbm.at[c]

        def _scale(buf, j):
            @pl.loop(0, 8)
            def _rowgrp(q):
                w16 = wb_v[j, pl.ds(q * 16, 16)]
                for i in range(16):
                    wv = lax.broadcast(w16[i], (16,))
                    r = q * 16 + i
                    for kk in range(DHALF // 16):
                        sl = pl.ds(kk * 16, 16)
                        buf[r, sl] = buf[r, sl] * wv

        rbase = s * GB
        NBLK = GB // 8

        @pl.loop(0, NBLK)
        def _blk(b):
            roff = rbase + b * 8
            pltpu.sync_copy(src_hbm.at[pl.ds(roff, 8)], sc_v)
            pltpu.sync_copy(dst_hbm.at[pl.ds(roff, 8)], dd_v)
            pltpu.sync_copy(w_hbm.at[pl.ds(roff, 8)], wb_v)
            pltpu.async_copy(h_half.at[sc_v.at[0]], rows0, sem0)
            for j in range(8):
                cur, sem_c = (rows0, sem0) if j % 2 == 0 else (rows1, sem1)
                nxt, sem_n = (rows1, sem1) if j % 2 == 0 else (rows0, sem0)
                pltpu.make_async_copy(h_half.at[sc_v.at[j]], cur, sem_c).wait()
                if j + 1 < 8:
                    pltpu.async_copy(h_half.at[sc_v.at[j + 1]], nxt, sem_n)
                _scale(cur, j)
                pltpu.sync_copy(cur, out_sh.at[dd_v.at[j]], add=True)

        plsc.subcore_barrier()
        for t in range(NTS // 128):
            r0 = s * NTS + t * 128
            pltpu.sync_copy(out_sh.at[pl.ds(r0, 128)], rows0)
            pltpu.sync_copy(rows0, out_hbm.at[c].at[pl.ds(r0, 128)])

    return k(src2d, dst2d, w2d, h_split)


def kernel(x, edge_index, W0, a_src0, a_dst0, b0, W1, a_src1, a_dst1, b1,
           W2, a_src2, a_dst2, b2, W_out, b_out):
    xp = jnp.pad(x, ((0, NP - N), (0, 0)))
    src = edge_index[0]
    dst = edge_index[1]
    npad = EP - E
    pad_idx = N + (jnp.arange(npad, dtype=jnp.int32) % (NP - N))
    src2d = jnp.concatenate([src, pad_idx]).reshape(ER, 128)
    dst2d = jnp.concatenate([dst, pad_idx]).reshape(ER, 128)

    params = [
        (W0, a_src0, a_dst0, b0),
        (W1, a_src1, a_dst1, b1),
        (W2, a_src2, a_dst2, b2),
    ]

    aggs = []
    prev_split = None
    for l, (W, a_s, a_d, b) in enumerate(params):
        a2 = jnp.stack([a_s, a_d])
        if l == 0:
            h_split, asad = _tc_layer0(xp, W, a2)
        else:
            h_split, asad = _tc_layer(prev_split, params[l - 1][3][None, :], W, a2)
        exw2d, den2 = _sc_edge_softmax(src2d, dst2d, asad[0], asad[1])
        w2d = _sc_edge_weights(dst2d, exw2d, den2)
        agg = _sc_aggregate(src2d, dst2d, w2d, h_split)
        aggs.append(agg)
        prev_split = agg

    b3 = jnp.stack([b0, b1, b2])
    wo_pad = jnp.pad(W_out, ((0, 0), (0, 128 - NCLS)))
    bo_pad = jnp.pad(b_out, (0, 128 - NCLS))[None, :]
    out = _tc_final(aggs[0], aggs[1], aggs[2], b3, wo_pad, bo_pad)
    return out[:N, :NCLS]


# speedup vs baseline: 28.7899x; 1.6980x over previous
import functools

import jax
import jax.numpy as jnp
from jax import lax
from jax.experimental import pallas as pl
from jax.experimental.pallas import tpu as pltpu
from jax.experimental.pallas import tpu_sc as plsc

N = 10000
NP = 10240
E = 320000
EP = 327680
DH = 256
DHALF = 128
NCLS = 40

NC = 2
NS = 16
NW = NC * NS

ER = EP // 128


BN = 1024


def _elu(x):
    return jnp.where(x > 0, x, jnp.exp(x) - 1.0)


def _tc_layer0_body(x_ref, w_ref, a2_ref, hs_ref, asad_ref):
    h = jnp.dot(x_ref[...], w_ref[...], preferred_element_type=jnp.float32)
    hs_ref[0] = h[:, :DHALF]
    hs_ref[1] = h[:, DHALF:]
    asad_ref[...] = lax.dot_general(a2_ref[...], h, (((1,), (1,)), ((), ())))


def _tc_layer_body(p_ref, b_ref, w_ref, a2_ref, hs_ref, asad_ref):
    xin = _elu(jnp.concatenate([p_ref[0], p_ref[1]], axis=-1) + b_ref[...])
    h = jnp.dot(xin, w_ref[...], preferred_element_type=jnp.float32)
    hs_ref[0] = h[:, :DHALF]
    hs_ref[1] = h[:, DHALF:]
    asad_ref[...] = lax.dot_general(a2_ref[...], h, (((1,), (1,)), ((), ())))


_TC_OUT_SHAPE = [
    jax.ShapeDtypeStruct((2, NP, DHALF), jnp.float32),
    jax.ShapeDtypeStruct((2, NP), jnp.float32),
]
_TC_OUT_SPECS = [
    pl.BlockSpec((2, BN, DHALF), lambda i: (0, i, 0)),
    pl.BlockSpec((2, BN), lambda i: (0, i)),
]


def _tc_layer0(xp, W, a2):
    return pl.pallas_call(
        _tc_layer0_body,
        grid=(NP // BN,),
        in_specs=[
            pl.BlockSpec((BN, 128), lambda i: (i, 0)),
            pl.BlockSpec((128, DH), lambda i: (0, 0)),
            pl.BlockSpec((2, DH), lambda i: (0, 0)),
        ],
        out_specs=_TC_OUT_SPECS,
        out_shape=_TC_OUT_SHAPE,
    )(xp, W, a2)


def _tc_layer(prev_split, b_prev, W, a2):
    return pl.pallas_call(
        _tc_layer_body,
        grid=(NP // BN,),
        in_specs=[
            pl.BlockSpec((2, BN, DHALF), lambda i: (0, i, 0)),
            pl.BlockSpec((1, DH), lambda i: (0, 0)),
            pl.BlockSpec((DH, DH), lambda i: (0, 0)),
            pl.BlockSpec((2, DH), lambda i: (0, 0)),
        ],
        out_specs=_TC_OUT_SPECS,
        out_shape=_TC_OUT_SHAPE,
    )(prev_split, b_prev, W, a2)


def _tc_final_body(a0_ref, a1_ref, a2_ref, b3_ref, wo_ref, bo_ref, out_ref):
    parts = []
    for l, ar in enumerate((a0_ref, a1_ref, a2_ref)):
        xl = jnp.concatenate([ar[0], ar[1]], axis=-1) + b3_ref[pl.ds(l, 1), :]
        parts.append(_elu(xl))
    cat = jnp.concatenate(parts, axis=-1)
    out_ref[...] = (
        jnp.dot(cat, wo_ref[...], preferred_element_type=jnp.float32) + bo_ref[...]
    )


def _tc_final(agg0, agg1, agg2, b3, wo_pad, bo_pad):
    return pl.pallas_call(
        _tc_final_body,
        grid=(NP // BN,),
        in_specs=[
            pl.BlockSpec((2, BN, DHALF), lambda i: (0, i, 0)),
            pl.BlockSpec((2, BN, DHALF), lambda i: (0, i, 0)),
            pl.BlockSpec((2, BN, DHALF), lambda i: (0, i, 0)),
            pl.BlockSpec((3, DH), lambda i: (0, 0)),
            pl.BlockSpec((3 * DH, 128), lambda i: (0, 0)),
            pl.BlockSpec((1, 128), lambda i: (0, 0)),
        ],
        out_specs=pl.BlockSpec((BN, 128), lambda i: (i, 0)),
        out_shape=jax.ShapeDtypeStruct((NP, 128), jnp.float32),
    )(agg0, agg1, agg2, b3, wo_pad, bo_pad)


_MESH = plsc.VectorSubcoreMesh(core_axis_name="c", subcore_axis_name="s")

EW_A = EP // NW
NSUB = 8
NCHA = EW_A // (NSUB * 128)
NTS = NP // NS


def _sc_edge_softmax(src2d, dst2d, as_arr, ad_arr):
    @functools.partial(
        pl.kernel,
        out_type=[
            jax.ShapeDtypeStruct((ER, 128), jnp.float32),
            jax.ShapeDtypeStruct((NC * NP,), jnp.float32),
        ],
        mesh=_MESH,
        scratch_types=[
            pltpu.VMEM((4, NSUB, 128), jnp.int32),
            pltpu.VMEM((4, NSUB, 128), jnp.int32),
            pltpu.VMEM((2, NSUB, 128), jnp.float32),
            pltpu.VMEM((2, NSUB, 128), jnp.float32),
            pltpu.VMEM((2, NSUB, 128), jnp.float32),
            pltpu.VMEM((NTS,), jnp.float32),
            pltpu.VMEM_SHARED((NP,), jnp.float32),
            pltpu.SemaphoreType.DMA,
            pltpu.SemaphoreType.DMA,
            pltpu.SemaphoreType.DMA,
            pltpu.SemaphoreType.DMA,
            pltpu.SemaphoreType.DMA,
        ],
    )
    def k(src_hbm, dst_hbm, as_hbm, ad_hbm, exw_hbm, den_hbm,
          src_v, dst_v, as_v, ad_v, ex_v, stage_v, den_sh,
          lsem, gsem0, gsem1, ssem0, ssem1):
        c = lax.axis_index("c")
        s = lax.axis_index("s")
        wid = c * NS + s

        @pl.loop(0, NTS // 16)
        def _zero(i):
            stage_v[pl.ds(i * 16, 16)] = jnp.zeros((16,), jnp.float32)

        pltpu.sync_copy(stage_v, den_sh.at[pl.ds(s * NTS, NTS)])
        plsc.subcore_barrier()

        rbase = wid * (EW_A // 128)
        gsems = (gsem0, gsem1)
        ssems = (ssem0, ssem1)

        def _issue_loads(g):
            roff = rbase + g * NSUB
            pltpu.async_copy(src_hbm.at[pl.ds(roff, NSUB)], src_v.at[g % 4], lsem)
            pltpu.async_copy(dst_hbm.at[pl.ds(roff, NSUB)], dst_v.at[g % 4], lsem)

        def _wait_loads(g):
            pltpu.make_async_copy(src_hbm.at[pl.ds(0, NSUB)], src_v.at[g % 4], lsem).wait()
            pltpu.make_async_copy(dst_hbm.at[pl.ds(0, NSUB)], dst_v.at[g % 4], lsem).wait()

        def _issue_gathers(g):
            for j in range(NSUB):
                pltpu.async_copy(as_hbm.at[src_v.at[g % 4].at[j]],
                                 as_v.at[g % 2].at[j], gsems[g % 2])
                pltpu.async_copy(ad_hbm.at[dst_v.at[g % 4].at[j]],
                                 ad_v.at[g % 2].at[j], gsems[g % 2])

        def _wait_gathers(g):
            for j in range(NSUB):
                pltpu.make_async_copy(as_hbm.at[src_v.at[g % 4].at[j]],
                                      as_v.at[g % 2].at[j], gsems[g % 2]).wait()
                pltpu.make_async_copy(ad_hbm.at[dst_v.at[g % 4].at[j]],
                                      ad_v.at[g % 2].at[j], gsems[g % 2]).wait()

        def _drain_scatters(g):
            for j in range(NSUB):
                pltpu.make_async_copy(ex_v.at[g % 2].at[j],
                                      den_sh.at[dst_v.at[g % 4].at[j]],
                                      ssems[g % 2]).wait()

        _issue_loads(0)
        _wait_loads(0)
        _issue_gathers(0)
        _issue_loads(1)

        for g in range(NCHA):
            if g + 1 < NCHA:
                _wait_loads(g + 1)
                _issue_gathers(g + 1)
                if g + 2 < NCHA:
                    _issue_loads(g + 2)
            _wait_gathers(g)
            if g >= 2:
                _drain_scatters(g - 2)

            @pl.loop(0, NSUB)
            def _vec(j):
                for kk in range(8):
                    sl = pl.ds(kk * 16, 16)
                    e = as_v[g % 2, j, sl] + ad_v[g % 2, j, sl]
                    e = jnp.where(e >= 0.0, e, e * jnp.float32(0.2))
                    ex_v[g % 2, j, sl] = jnp.exp(e)

            pltpu.sync_copy(ex_v.at[g % 2], exw_hbm.at[pl.ds(rbase + g * NSUB, NSUB)])
            for j in range(NSUB):
                pltpu.async_copy(ex_v.at[g % 2].at[j],
                                 den_sh.at[dst_v.at[g % 4].at[j]],
                                 ssems[g % 2], add=True)

        _drain_scatters(NCHA - 2)
        _drain_scatters(NCHA - 1)

        plsc.subcore_barrier()
        pltpu.sync_copy(den_sh.at[pl.ds(s * NTS, NTS)], stage_v)
        pltpu.sync_copy(stage_v, den_hbm.at[pl.ds(c * NP + s * NTS, NTS)])

    return k(src2d, dst2d, as_arr, ad_arr)


EW_B = EP // NS
GB = EW_B // 128


def _sc_aggregate(src2d, dst2d, exw2d, den2, h_split):
    @functools.partial(
        pl.kernel,
        out_type=jax.ShapeDtypeStruct((2, NP, DHALF), jnp.float32),
        mesh=_MESH,
        scratch_types=[
            pltpu.VMEM((8, 128), jnp.int32),
            pltpu.VMEM((8, 128), jnp.int32),
            pltpu.VMEM((8, 128), jnp.int32),
            pltpu.VMEM((8, 128), jnp.int32),
            pltpu.VMEM((8, 128), jnp.float32),
            pltpu.VMEM((8, 128), jnp.float32),
            pltpu.VMEM((128, DHALF), jnp.float32),
            pltpu.VMEM((128, DHALF), jnp.float32),
            pltpu.VMEM((NTS,), jnp.float32),
            pltpu.VMEM((NTS,), jnp.float32),
            pltpu.VMEM_SHARED((NP, DHALF), jnp.float32),
            pltpu.SemaphoreType.DMA,
            pltpu.SemaphoreType.DMA,
            pltpu.SemaphoreType.DMA,
            pltpu.SemaphoreType.DMA,
            pltpu.SemaphoreType.DMA,
        ],
    )
    def k(src_hbm, dst_hbm, w_hbm, den_hbm, hs_hbm, out_hbm,
          sc0, sc1, dd0, dd1, wb0, wb1, rows0, rows1,
          recip_v, dtmp_v, out_sh, isem, gsem0, gsem1, ssem0, ssem1):
        c = lax.axis_index("c")
        s = lax.axis_index("s")

        @pl.loop(0, 128)
        def _zr(r):
            for kk in range(DHALF // 16):
                rows0[r, pl.ds(kk * 16, 16)] = jnp.zeros((16,), jnp.float32)

        for t in range(NTS // 128):
            pltpu.sync_copy(rows0, out_sh.at[pl.ds(s * NTS + t * 128, 128)])

        pltpu.sync_copy(den_hbm.at[pl.ds(s * NTS, NTS)], recip_v)
        pltpu.sync_copy(den_hbm.at[pl.ds(NP + s * NTS, NTS)], dtmp_v)

        @pl.loop(0, NTS // 16)
        def _rcp(i):
            sl = pl.ds(i * 16, 16)
            recip_v[sl] = jnp.float32(1.0) / (
                recip_v[sl] + dtmp_v[sl] + jnp.float32(1e-16))

        plsc.subcore_barrier()

        h_half = hs_hbm.at[c]
        rowbufs = (rows0, rows1)
        gsems = (gsem0, gsem1)
        ssems = (ssem0, ssem1)

        def _scale(buf, wb, j):
            @pl.loop(0, 8)
            def _rowgrp(q):
                w16 = wb[j, pl.ds(q * 16, 16)]
                for i in range(16):
                    wv = lax.broadcast(w16[i], (16,))
                    r = q * 16 + i
                    for kk in range(DHALF // 16):
                        sl = pl.ds(kk * 16, 16)
                        buf[r, sl] = buf[r, sl] * wv

        rbase = s * GB
        NBLK = GB // 8

        def _issue_iblock(roff, dsc, ddd, dwb):
            pltpu.async_copy(src_hbm.at[pl.ds(roff, 8)], dsc, isem)
            pltpu.async_copy(dst_hbm.at[pl.ds(roff, 8)], ddd, isem)
            pltpu.async_copy(w_hbm.at[pl.ds(roff, 8)], dwb, isem)

        def _wait_iblock(dsc, ddd, dwb):
            pltpu.make_async_copy(src_hbm.at[pl.ds(0, 8)], dsc, isem).wait()
            pltpu.make_async_copy(dst_hbm.at[pl.ds(0, 8)], ddd, isem).wait()
            pltpu.make_async_copy(w_hbm.at[pl.ds(0, 8)], dwb, isem).wait()

        _issue_iblock(rbase, sc0, dd0, wb0)
        _wait_iblock(sc0, dd0, wb0)
        pltpu.async_copy(h_half.at[sc0.at[0]], rows0, gsem0)

        def _block(b, not_first, has_next, cur_sc, cur_dd, cur_wb,
                   nxt_sc, nxt_dd, nxt_wb):
            for j in range(8):
                jp = j % 2
                cur, gsem_c, ssem_c = rowbufs[jp], gsems[jp], ssems[jp]
                nxt, gsem_n, ssem_n = rowbufs[1 - jp], gsems[1 - jp], ssems[1 - jp]
                pltpu.make_async_copy(h_half.at[cur_sc.at[j]], cur, gsem_c).wait()
                if j > 0:
                    pltpu.make_async_copy(
                        nxt, out_sh.at[cur_dd.at[j - 1]], ssem_n).wait()
                else:
                    @pl.when(not_first)
                    def _():
                        pltpu.make_async_copy(
                            nxt, out_sh.at[cur_dd.at[7]], ssem_n).wait()
                    @pl.when(has_next)
                    def _():
                        _issue_iblock(rbase + (b + 1) * 8, nxt_sc, nxt_dd, nxt_wb)
                if j + 1 < 8:
                    pltpu.async_copy(h_half.at[cur_sc.at[j + 1]], nxt, gsem_n)
                else:
                    @pl.when(has_next)
                    def _():
                        _wait_iblock(nxt_sc, nxt_dd, nxt_wb)
                        pltpu.async_copy(h_half.at[nxt_sc.at[0]], nxt, gsem_n)
                _scale(cur, cur_wb, j)
                pltpu.async_copy(cur, out_sh.at[cur_dd.at[j]], ssem_c, add=True)

        @pl.loop(0, NBLK, step=2)
        def _blk(b):
            _block(b, b > 0, True, sc0, dd0, wb0, sc1, dd1, wb1)
            _block(b + 1, True, b + 2 < NBLK, sc1, dd1, wb1, sc0, dd0, wb0)

        pltpu.make_async_copy(rows1, out_sh.at[dd0.at[0]], ssem1).wait()

        plsc.subcore_barrier()

        for t in range(NTS // 128):
            r0 = s * NTS + t * 128
            pltpu.sync_copy(out_sh.at[pl.ds(r0, 128)], rows0)

            @pl.loop(0, 8)
            def _nrm(q):
                r16 = recip_v[pl.ds(t * 128 + q * 16, 16)]
                for i in range(16):
                    rv = lax.broadcast(r16[i], (16,))
                    r = q * 16 + i
                    for kk in range(DHALF // 16):
                        sl = pl.ds(kk * 16, 16)
                        rows0[r, sl] = rows0[r, sl] * rv

            pltpu.sync_copy(rows0, out_hbm.at[c].at[pl.ds(r0, 128)])

    return k(src2d, dst2d, exw2d, den2, h_split)


def kernel(x, edge_index, W0, a_src0, a_dst0, b0, W1, a_src1, a_dst1, b1,
           W2, a_src2, a_dst2, b2, W_out, b_out):
    xp = jnp.pad(x, ((0, NP - N), (0, 0)))
    src = edge_index[0]
    dst = edge_index[1]
    npad = EP - E
    pad_idx = N + (jnp.arange(npad, dtype=jnp.int32) % (NP - N))
    src2d = jnp.concatenate([src, pad_idx]).reshape(ER, 128)
    dst2d = jnp.concatenate([dst, pad_idx]).reshape(ER, 128)

    params = [
        (W0, a_src0, a_dst0, b0),
        (W1, a_src1, a_dst1, b1),
        (W2, a_src2, a_dst2, b2),
    ]

    aggs = []
    prev_split = None
    for l, (W, a_s, a_d, b) in enumerate(params):
        a2 = jnp.stack([a_s, a_d])
        if l == 0:
            h_split, asad = _tc_layer0(xp, W, a2)
        else:
            h_split, asad = _tc_layer(prev_split, params[l - 1][3][None, :], W, a2)
        exw2d, den2 = _sc_edge_softmax(src2d, dst2d, asad[0], asad[1])
        agg = _sc_aggregate(src2d, dst2d, exw2d, den2, h_split)
        aggs.append(agg)
        prev_split = agg

    b3 = jnp.stack([b0, b1, b2])
    wo_pad = jnp.pad(W_out, ((0, 0), (0, 128 - NCLS)))
    bo_pad = jnp.pad(b_out, (0, 128 - NCLS))[None, :]
    out = _tc_final(aggs[0], aggs[1], aggs[2], b3, wo_pad, bo_pad)
    return out[:N, :NCLS]


# speedup vs baseline: 34.3492x; 1.1931x over previous
import functools

import jax
import jax.numpy as jnp
from jax import lax
from jax.experimental import pallas as pl
from jax.experimental.pallas import tpu as pltpu
from jax.experimental.pallas import tpu_sc as plsc

N = 10000
NP = 10240
E = 320000
EP = 327680
DH = 256
DHALF = 128
NCLS = 40

NC = 2
NS = 16
NW = NC * NS

ER = EP // 128


BN = 1024


def _elu(x):
    return jnp.where(x > 0, x, jnp.exp(x) - 1.0)


def _tc_layer0_body(x_ref, w_ref, a2_ref, hs_ref, asad_ref):
    h = jnp.dot(x_ref[...], w_ref[...], preferred_element_type=jnp.float32)
    hs_ref[0] = h[:, :DHALF]
    hs_ref[1] = h[:, DHALF:]
    asad_ref[...] = lax.dot_general(a2_ref[...], h, (((1,), (1,)), ((), ())))


def _tc_layer_body(p_ref, b_ref, w_ref, a2_ref, hs_ref, asad_ref):
    xin = _elu(jnp.concatenate([p_ref[0], p_ref[1]], axis=-1) + b_ref[...])
    h = jnp.dot(xin, w_ref[...], preferred_element_type=jnp.float32)
    hs_ref[0] = h[:, :DHALF]
    hs_ref[1] = h[:, DHALF:]
    asad_ref[...] = lax.dot_general(a2_ref[...], h, (((1,), (1,)), ((), ())))


_TC_OUT_SHAPE = [
    jax.ShapeDtypeStruct((2, NP, DHALF), jnp.float32),
    jax.ShapeDtypeStruct((2, NP), jnp.float32),
]
_TC_OUT_SPECS = [
    pl.BlockSpec((2, BN, DHALF), lambda i: (0, i, 0)),
    pl.BlockSpec((2, BN), lambda i: (0, i)),
]


def _tc_layer0(xp, W, a2):
    return pl.pallas_call(
        _tc_layer0_body,
        grid=(NP // BN,),
        in_specs=[
            pl.BlockSpec((BN, 128), lambda i: (i, 0)),
            pl.BlockSpec((128, DH), lambda i: (0, 0)),
            pl.BlockSpec((2, DH), lambda i: (0, 0)),
        ],
        out_specs=_TC_OUT_SPECS,
        out_shape=_TC_OUT_SHAPE,
    )(xp, W, a2)


def _tc_layer(prev_split, b_prev, W, a2):
    return pl.pallas_call(
        _tc_layer_body,
        grid=(NP // BN,),
        in_specs=[
            pl.BlockSpec((2, BN, DHALF), lambda i: (0, i, 0)),
            pl.BlockSpec((1, DH), lambda i: (0, 0)),
            pl.BlockSpec((DH, DH), lambda i: (0, 0)),
            pl.BlockSpec((2, DH), lambda i: (0, 0)),
        ],
        out_specs=_TC_OUT_SPECS,
        out_shape=_TC_OUT_SHAPE,
    )(prev_split, b_prev, W, a2)


def _tc_final_body(a0_ref, a1_ref, a2_ref, b3_ref, wo_ref, bo_ref, out_ref):
    parts = []
    for l, ar in enumerate((a0_ref, a1_ref, a2_ref)):
        xl = jnp.concatenate([ar[0], ar[1]], axis=-1) + b3_ref[pl.ds(l, 1), :]
        parts.append(_elu(xl))
    cat = jnp.concatenate(parts, axis=-1)
    out_ref[...] = (
        jnp.dot(cat, wo_ref[...], preferred_element_type=jnp.float32) + bo_ref[...]
    )


def _tc_final(agg0, agg1, agg2, b3, wo_pad, bo_pad):
    return pl.pallas_call(
        _tc_final_body,
        grid=(NP // BN,),
        in_specs=[
            pl.BlockSpec((2, BN, DHALF), lambda i: (0, i, 0)),
            pl.BlockSpec((2, BN, DHALF), lambda i: (0, i, 0)),
            pl.BlockSpec((2, BN, DHALF), lambda i: (0, i, 0)),
            pl.BlockSpec((3, DH), lambda i: (0, 0)),
            pl.BlockSpec((3 * DH, 128), lambda i: (0, 0)),
            pl.BlockSpec((1, 128), lambda i: (0, 0)),
        ],
        out_specs=pl.BlockSpec((BN, 128), lambda i: (i, 0)),
        out_shape=jax.ShapeDtypeStruct((NP, 128), jnp.float32),
    )(agg0, agg1, agg2, b3, wo_pad, bo_pad)


_MESH = plsc.VectorSubcoreMesh(core_axis_name="c", subcore_axis_name="s")

EW_A = EP // NW
NSUB = 8
NCHA = EW_A // (NSUB * 128)
NTS = NP // NS


def _sc_edge_softmax(src2d, dst2d, as_arr, ad_arr):
    @functools.partial(
        pl.kernel,
        out_type=[
            jax.ShapeDtypeStruct((ER, 128), jnp.float32),
            jax.ShapeDtypeStruct((NC * NP,), jnp.float32),
        ],
        mesh=_MESH,
        scratch_types=[
            pltpu.VMEM((4, NSUB, 128), jnp.int32),
            pltpu.VMEM((4, NSUB, 128), jnp.int32),
            pltpu.VMEM((2, NSUB, 128), jnp.float32),
            pltpu.VMEM((2, NSUB, 128), jnp.float32),
            pltpu.VMEM((2, NSUB, 128), jnp.float32),
            pltpu.VMEM((NTS,), jnp.float32),
            pltpu.VMEM_SHARED((NP,), jnp.float32),
            pltpu.VMEM_SHARED((NP,), jnp.float32),
            pltpu.VMEM_SHARED((NP,), jnp.float32),
            pltpu.SemaphoreType.DMA,
            pltpu.SemaphoreType.DMA,
            pltpu.SemaphoreType.DMA,
            pltpu.SemaphoreType.DMA,
            pltpu.SemaphoreType.DMA,
        ],
    )
    def k(src_hbm, dst_hbm, as_hbm, ad_hbm, exw_hbm, den_hbm,
          src_v, dst_v, as_v, ad_v, ex_v, stage_v, den_sh, as_sh, ad_sh,
          lsem, gsem0, gsem1, ssem0, ssem1):
        c = lax.axis_index("c")
        s = lax.axis_index("s")
        wid = c * NS + s

        @pl.loop(0, NTS // 16)
        def _zero(i):
            stage_v[pl.ds(i * 16, 16)] = jnp.zeros((16,), jnp.float32)

        pltpu.sync_copy(stage_v, den_sh.at[pl.ds(s * NTS, NTS)])
        sl_me = pl.ds(s * NTS, NTS)
        pltpu.sync_copy(as_hbm.at[sl_me], stage_v)
        pltpu.sync_copy(stage_v, as_sh.at[sl_me])
        pltpu.sync_copy(ad_hbm.at[sl_me], stage_v)
        pltpu.sync_copy(stage_v, ad_sh.at[sl_me])
        plsc.subcore_barrier()

        rbase = wid * (EW_A // 128)
        gsems = (gsem0, gsem1)
        ssems = (ssem0, ssem1)

        def _issue_loads(g):
            roff = rbase + g * NSUB
            pltpu.async_copy(src_hbm.at[pl.ds(roff, NSUB)], src_v.at[g % 4], lsem)
            pltpu.async_copy(dst_hbm.at[pl.ds(roff, NSUB)], dst_v.at[g % 4], lsem)

        def _wait_loads(g):
            pltpu.make_async_copy(src_hbm.at[pl.ds(0, NSUB)], src_v.at[g % 4], lsem).wait()
            pltpu.make_async_copy(dst_hbm.at[pl.ds(0, NSUB)], dst_v.at[g % 4], lsem).wait()

        def _issue_gathers(g):
            for j in range(NSUB):
                pltpu.async_copy(as_sh.at[src_v.at[g % 4].at[j]],
                                 as_v.at[g % 2].at[j], gsems[g % 2])
                pltpu.async_copy(ad_sh.at[dst_v.at[g % 4].at[j]],
                                 ad_v.at[g % 2].at[j], gsems[g % 2])

        def _wait_gathers(g):
            for j in range(NSUB):
                pltpu.make_async_copy(as_sh.at[src_v.at[g % 4].at[j]],
                                      as_v.at[g % 2].at[j], gsems[g % 2]).wait()
                pltpu.make_async_copy(ad_sh.at[dst_v.at[g % 4].at[j]],
                                      ad_v.at[g % 2].at[j], gsems[g % 2]).wait()

        def _drain_scatters(g):
            for j in range(NSUB):
                pltpu.make_async_copy(ex_v.at[g % 2].at[j],
                                      den_sh.at[dst_v.at[g % 4].at[j]],
                                      ssems[g % 2]).wait()

        _issue_loads(0)
        _wait_loads(0)
        _issue_gathers(0)
        _issue_loads(1)

        for g in range(NCHA):
            if g + 1 < NCHA:
                _wait_loads(g + 1)
                _issue_gathers(g + 1)
                if g + 2 < NCHA:
                    _issue_loads(g + 2)
            _wait_gathers(g)
            if g >= 2:
                _drain_scatters(g - 2)

            @pl.loop(0, NSUB)
            def _vec(j):
                for kk in range(8):
                    sl = pl.ds(kk * 16, 16)
                    e = as_v[g % 2, j, sl] + ad_v[g % 2, j, sl]
                    e = jnp.where(e >= 0.0, e, e * jnp.float32(0.2))
                    ex_v[g % 2, j, sl] = jnp.exp(e)

            pltpu.sync_copy(ex_v.at[g % 2], exw_hbm.at[pl.ds(rbase + g * NSUB, NSUB)])
            for j in range(NSUB):
                pltpu.async_copy(ex_v.at[g % 2].at[j],
                                 den_sh.at[dst_v.at[g % 4].at[j]],
                                 ssems[g % 2], add=True)

        _drain_scatters(NCHA - 2)
        _drain_scatters(NCHA - 1)

        plsc.subcore_barrier()
        pltpu.sync_copy(den_sh.at[pl.ds(s * NTS, NTS)], stage_v)
        pltpu.sync_copy(stage_v, den_hbm.at[pl.ds(c * NP + s * NTS, NTS)])

    return k(src2d, dst2d, as_arr, ad_arr)


EW_B = EP // NS
GB = EW_B // 128


def _sc_aggregate(src2d, dst2d, exw2d, den2, h_split):
    @functools.partial(
        pl.kernel,
        out_type=jax.ShapeDtypeStruct((2, NP, DHALF), jnp.float32),
        mesh=_MESH,
        scratch_types=[
            pltpu.VMEM((8, 128), jnp.int32),
            pltpu.VMEM((8, 128), jnp.int32),
            pltpu.VMEM((8, 128), jnp.int32),
            pltpu.VMEM((8, 128), jnp.int32),
            pltpu.VMEM((8, 128), jnp.float32),
            pltpu.VMEM((8, 128), jnp.float32),
            pltpu.VMEM((128, DHALF), jnp.float32),
            pltpu.VMEM((128, DHALF), jnp.float32),
            pltpu.VMEM((NTS,), jnp.float32),
            pltpu.VMEM((NTS,), jnp.float32),
            pltpu.VMEM_SHARED((NP, DHALF), jnp.float32),
            pltpu.SemaphoreType.DMA,
            pltpu.SemaphoreType.DMA,
            pltpu.SemaphoreType.DMA,
            pltpu.SemaphoreType.DMA,
            pltpu.SemaphoreType.DMA,
        ],
    )
    def k(src_hbm, dst_hbm, w_hbm, den_hbm, hs_hbm, out_hbm,
          sc0, sc1, dd0, dd1, wb0, wb1, rows0, rows1,
          recip_v, dtmp_v, out_sh, isem, gsem0, gsem1, ssem0, ssem1):
        c = lax.axis_index("c")
        s = lax.axis_index("s")

        @pl.loop(0, 128)
        def _zr(r):
            for kk in range(DHALF // 16):
                rows0[r, pl.ds(kk * 16, 16)] = jnp.zeros((16,), jnp.float32)

        for t in range(NTS // 128):
            pltpu.sync_copy(rows0, out_sh.at[pl.ds(s * NTS + t * 128, 128)])

        pltpu.sync_copy(den_hbm.at[pl.ds(s * NTS, NTS)], recip_v)
        pltpu.sync_copy(den_hbm.at[pl.ds(NP + s * NTS, NTS)], dtmp_v)

        @pl.loop(0, NTS // 16)
        def _rcp(i):
            sl = pl.ds(i * 16, 16)
            recip_v[sl] = jnp.float32(1.0) / (
                recip_v[sl] + dtmp_v[sl] + jnp.float32(1e-16))

        plsc.subcore_barrier()

        h_half = hs_hbm.at[c]
        rowbufs = (rows0, rows1)
        gsems = (gsem0, gsem1)
        ssems = (ssem0, ssem1)

        def _scale(buf, wb, j):
            @pl.loop(0, 8)
            def _rowgrp(q):
                w16 = wb[j, pl.ds(q * 16, 16)]
                for i in range(16):
                    wv = lax.broadcast(w16[i], (16,))
                    r = q * 16 + i
                    for kk in range(DHALF // 16):
                        sl = pl.ds(kk * 16, 16)
                        buf[r, sl] = buf[r, sl] * wv

        rbase = s * GB
        NBLK = GB // 8

        def _issue_iblock(roff, dsc, ddd, dwb):
            pltpu.async_copy(src_hbm.at[pl.ds(roff, 8)], dsc, isem)
            pltpu.async_copy(dst_hbm.at[pl.ds(roff, 8)], ddd, isem)
            pltpu.async_copy(w_hbm.at[pl.ds(roff, 8)], dwb, isem)

        def _wait_iblock(dsc, ddd, dwb):
            pltpu.make_async_copy(src_hbm.at[pl.ds(0, 8)], dsc, isem).wait()
            pltpu.make_async_copy(dst_hbm.at[pl.ds(0, 8)], ddd, isem).wait()
            pltpu.make_async_copy(w_hbm.at[pl.ds(0, 8)], dwb, isem).wait()

        _issue_iblock(rbase, sc0, dd0, wb0)
        _wait_iblock(sc0, dd0, wb0)
        pltpu.async_copy(h_half.at[sc0.at[0]], rows0, gsem0)

        def _block(b, not_first, has_next, cur_sc, cur_dd, cur_wb,
                   nxt_sc, nxt_dd, nxt_wb):
            for j in range(8):
                jp = j % 2
                cur, gsem_c, ssem_c = rowbufs[jp], gsems[jp], ssems[jp]
                nxt, gsem_n, ssem_n = rowbufs[1 - jp], gsems[1 - jp], ssems[1 - jp]
                pltpu.make_async_copy(h_half.at[cur_sc.at[j]], cur, gsem_c).wait()
                if j > 0:
                    pltpu.make_async_copy(
                        nxt, out_sh.at[cur_dd.at[j - 1]], ssem_n).wait()
                else:
                    @pl.when(not_first)
                    def _():
                        pltpu.make_async_copy(
                            nxt, out_sh.at[cur_dd.at[7]], ssem_n).wait()
                    @pl.when(has_next)
                    def _():
                        _issue_iblock(rbase + (b + 1) * 8, nxt_sc, nxt_dd, nxt_wb)
                if j + 1 < 8:
                    pltpu.async_copy(h_half.at[cur_sc.at[j + 1]], nxt, gsem_n)
                else:
                    @pl.when(has_next)
                    def _():
                        _wait_iblock(nxt_sc, nxt_dd, nxt_wb)
                        pltpu.async_copy(h_half.at[nxt_sc.at[0]], nxt, gsem_n)
                _scale(cur, cur_wb, j)
                pltpu.async_copy(cur, out_sh.at[cur_dd.at[j]], ssem_c, add=True)

        @pl.loop(0, NBLK, step=2)
        def _blk(b):
            _block(b, b > 0, True, sc0, dd0, wb0, sc1, dd1, wb1)
            _block(b + 1, True, b + 2 < NBLK, sc1, dd1, wb1, sc0, dd0, wb0)

        pltpu.make_async_copy(rows1, out_sh.at[dd0.at[0]], ssem1).wait()

        plsc.subcore_barrier()

        for t in range(NTS // 128):
            r0 = s * NTS + t * 128
            pltpu.sync_copy(out_sh.at[pl.ds(r0, 128)], rows0)

            @pl.loop(0, 8)
            def _nrm(q):
                r16 = recip_v[pl.ds(t * 128 + q * 16, 16)]
                for i in range(16):
                    rv = lax.broadcast(r16[i], (16,))
                    r = q * 16 + i
                    for kk in range(DHALF // 16):
                        sl = pl.ds(kk * 16, 16)
                        rows0[r, sl] = rows0[r, sl] * rv

            pltpu.sync_copy(rows0, out_hbm.at[c].at[pl.ds(r0, 128)])

    return k(src2d, dst2d, exw2d, den2, h_split)


def kernel(x, edge_index, W0, a_src0, a_dst0, b0, W1, a_src1, a_dst1, b1,
           W2, a_src2, a_dst2, b2, W_out, b_out):
    xp = jnp.pad(x, ((0, NP - N), (0, 0)))
    src = edge_index[0]
    dst = edge_index[1]
    npad = EP - E
    pad_idx = N + (jnp.arange(npad, dtype=jnp.int32) % (NP - N))
    src2d = jnp.concatenate([src, pad_idx]).reshape(ER, 128)
    dst2d = jnp.concatenate([dst, pad_idx]).reshape(ER, 128)

    params = [
        (W0, a_src0, a_dst0, b0),
        (W1, a_src1, a_dst1, b1),
        (W2, a_src2, a_dst2, b2),
    ]

    aggs = []
    prev_split = None
    for l, (W, a_s, a_d, b) in enumerate(params):
        a2 = jnp.stack([a_s, a_d])
        if l == 0:
            h_split, asad = _tc_layer0(xp, W, a2)
        else:
            h_split, asad = _tc_layer(prev_split, params[l - 1][3][None, :], W, a2)
        exw2d, den2 = _sc_edge_softmax(src2d, dst2d, asad[0], asad[1])
        agg = _sc_aggregate(src2d, dst2d, exw2d, den2, h_split)
        aggs.append(agg)
        prev_split = agg

    b3 = jnp.stack([b0, b1, b2])
    wo_pad = jnp.pad(W_out, ((0, 0), (0, 128 - NCLS)))
    bo_pad = jnp.pad(b_out, (0, 128 - NCLS))[None, :]
    out = _tc_final(aggs[0], aggs[1], aggs[2], b3, wo_pad, bo_pad)
    return out[:N, :NCLS]
